# baseline (speedup 1.0000x reference)
import functools

import jax
import jax.numpy as jnp
from jax import lax
from jax.experimental import pallas as pl
from jax.experimental.pallas import tpu as pltpu
from jax.experimental.pallas import tpu_sc as plsc

_D = 128
_NB = 64
_K = 32
_BBLK = 512
_NW = 32


def _shuf(x, perm):
    return lax.gather(
        x, perm[:, None],
        dimension_numbers=lax.GatherDimensionNumbers(
            offset_dims=(), collapsed_slice_dims=(0,), start_index_map=(0,)),
        slice_sizes=(1,),
        mode=lax.GatherScatterMode.PROMISE_IN_BOUNDS)


def _clean16(iota, k, v):
    for dist in (8, 4, 2, 1):
        perm = iota ^ dist
        pk, pv = _shuf(k, perm), _shuf(v, perm)
        up = (iota & dist) == 0
        c = k > pk
        bk = jnp.where(c, k, pk)
        bv = jnp.where(c, v, pv)
        sk = jnp.where(c, pk, k)
        sv = jnp.where(c, pv, v)
        k = jnp.where(up, bk, sk)
        v = jnp.where(up, bv, sv)
    return k, v


def _merge16(iota, ak, av, bk, bv):
    rbk, rbv = lax.rev(bk, (0,)), lax.rev(bv, (0,))
    c = ak > rbk
    hk = jnp.where(c, ak, rbk)
    hv = jnp.where(c, av, rbv)
    lk = jnp.where(c, rbk, ak)
    lv = jnp.where(c, rbv, av)
    hk, hv = _clean16(iota, hk, hv)
    lk, lv = _clean16(iota, lk, lv)
    return hk, hv, lk, lv


def _rsqrt_nr(x):
    xi = lax.bitcast_convert_type(x, jnp.int32)
    yi = jnp.int32(0x5F3759DF) - (xi >> 1)
    y = lax.bitcast_convert_type(yi, jnp.float32)
    for _ in range(3):
        y = y * (1.5 - 0.5 * x * y * y)
    return y


def _float_key(sim):
    b = lax.bitcast_convert_type(sim, jnp.int32)
    mask = b >> 31
    return b ^ (mask & jnp.int32(0x7FFFFFFF))


def _sc_neighbor_call(stage, emb, npad):
    npi = npad // _NW
    mesh = plsc.VectorSubcoreMesh(core_axis_name="c", subcore_axis_name="s")

    @functools.partial(
        pl.kernel, mesh=mesh,
        out_type=jax.ShapeDtypeStruct((npad, 2 * _D), jnp.float32),
        compiler_params=pltpu.CompilerParams(needs_layout_passes=False),
        scratch_types=(
            [pltpu.VMEM((144,), jnp.int32)] * 4 +
            [pltpu.VMEM((72, _D), jnp.float32)] * 4 +
            [pltpu.VMEM((_NB,), jnp.int32)] * 4 +
            [pltpu.VMEM((_NB, _D), jnp.float32)] * 4 +
            [
                pltpu.VMEM((_NB, 16), jnp.float32),
                pltpu.VMEM((_NB, 16), jnp.float32),
                pltpu.VMEM((2 * _D,), jnp.float32),
            ] +
            [pltpu.SemaphoreType.DMA] * 8
        ),
    )
    def sc_k(stage_hbm, emb_hbm, out_hbm,
             stage_0, stage_1, stage_2, stage_3,
             rows_0, rows_1, rows_2, rows_3,
             selidx_0, selidx_1, selidx_2, selidx_3,
             selrows_0, selrows_1, selrows_2, selrows_3,
             p_buf, q_buf, out_v,
             sem_r0, sem_r1, sem_r2, sem_r3,
             sem_s0, sem_s1, sem_s2, sem_s3):
        wid = lax.axis_index("s") * 2 + lax.axis_index("c")
        base = wid * npi
        iota = lax.iota(jnp.int32, 16)
        jvs = [iota + 16 * g for g in range(4)]
        zero = jnp.zeros((16,), jnp.float32)
        inv = jnp.float32(1.0 / _K)

        def _tree16(vecs):
            dist = 1
            while len(vecs) > 1:
                nxt = []
                for t in range(len(vecs) // 2):
                    a, b = vecs[2 * t], vecs[2 * t + 1]
                    sa = a + _shuf(a, iota ^ dist)
                    sb = b + _shuf(b, iota ^ dist)
                    nxt.append(jnp.where((iota & dist) == 0, sa, sb))
                vecs = nxt
                dist *= 2
            return vecs[0]

        def _sims_sort_sel(rows_v, stage_v, selidx_v):
            cv = jnp.full((16,), 64, jnp.int32)
            cen = [plsc.load_gather(rows_v, [cv, iota + cc * 16])
                   for cc in range(8)]

            def jbody(jo, carry):
                for ju in range(2):
                    j = jo * 2 + ju
                    jv = jnp.full((16,), j, jnp.int32)
                    e = [plsc.load_gather(rows_v, [jv, iota + cc * 16])
                         for cc in range(8)]
                    p = cen[0] * e[0]
                    q = e[0] * e[0]
                    for cc in range(1, 8):
                        p = p + cen[cc] * e[cc]
                        q = q + e[cc] * e[cc]
                    plsc.store_scatter(p_buf, [jv, iota], p)
                    plsc.store_scatter(q_buf, [jv, iota], q)
                return carry

            lax.fori_loop(0, _NB // 2, jbody, 0)

            keys, vals = [], []
            for g in range(4):
                dots = _tree16([p_buf[g * 16 + jj] for jj in range(16)])
                nrms = _tree16([q_buf[g * 16 + jj] for jj in range(16)])
                sim = dots * _rsqrt_nr(nrms)
                sk = _float_key(sim) - jvs[g]
                k_, v_ = plsc.sort_key_val(sk, jvs[g], descending=True)
                keys.append(k_)
                vals.append(v_)

            a0k, a0v, a1k, a1v = _merge16(iota, keys[0], vals[0],
                                          keys[1], vals[1])
            b0k, b0v, b1k, b1v = _merge16(iota, keys[2], vals[2],
                                          keys[3], vals[3])
            r0v = lax.rev(b1v, (0,))
            r1v = lax.rev(b0v, (0,))
            h0v = jnp.where(a0k > lax.rev(b1k, (0,)), a0v, r0v)
            h1v = jnp.where(a1k > lax.rev(b0k, (0,)), a1v, r1v)

            esel0 = plsc.load_gather(stage_v, [h0v])
            esel1 = plsc.load_gather(stage_v, [h1v])
            rsel0 = plsc.load_gather(stage_v, [h0v + 72])
            rsel1 = plsc.load_gather(stage_v, [h1v + 72])
            selidx_v[pl.ds(0, 16)] = esel0
            selidx_v[pl.ds(16, 16)] = esel1
            selidx_v[pl.ds(32, 16)] = rsel0
            selidx_v[pl.ds(48, 16)] = rsel1

        def _sum_store(item, selrows_v):
            def sbody2(jo, accs):
                new = list(accs)
                for ju in range(4):
                    jv = jnp.full((16,), jo * 4 + ju, jnp.int32)
                    for cc in range(8):
                        colidx = iota + cc * 16
                        ecol = plsc.load_gather(selrows_v, [jv, colidx])
                        rcol = plsc.load_gather(selrows_v, [jv + 32, colidx])
                        new[cc] = new[cc] + ecol
                        new[8 + cc] = new[8 + cc] + rcol
                return tuple(new)

            sums = lax.fori_loop(0, 8, sbody2, (zero,) * 16)
            for cc in range(8):
                out_v[pl.ds(cc * 16, 16)] = sums[8 + cc] * inv
                out_v[pl.ds(_D + cc * 16, 16)] = sums[cc] * inv
            pltpu.sync_copy(out_v, out_hbm.at[item])

        stages = (stage_0, stage_1, stage_2, stage_3)
        rows = (rows_0, rows_1, rows_2, rows_3)
        selidx = (selidx_0, selidx_1, selidx_2, selidx_3)
        selrows = (selrows_0, selrows_1, selrows_2, selrows_3)
        sem_r = (sem_r0, sem_r1, sem_r2, sem_r3)
        sem_s = (sem_s0, sem_s1, sem_s2, sem_s3)

        def quad_body(t, carry):
            items = [base + 4 * t + s for s in range(4)]
            cp_r = []
            for s in range(4):
                pltpu.sync_copy(stage_hbm.at[items[s]], stages[s])
                cp_r.append(pltpu.async_copy(
                    emb_hbm.at[stages[s].at[pl.ds(0, 72)]], rows[s],
                    sem_r[s]))
            cp_s = [None] * 4
            for s in range(4):
                cp_r[s].wait()
                _sims_sort_sel(rows[s], stages[s], selidx[s])
                cp_s[s] = pltpu.async_copy(
                    emb_hbm.at[selidx[s]], selrows[s], sem_s[s])
                if s >= 1:
                    cp_s[s - 1].wait()
                    _sum_store(items[s - 1], selrows[s - 1])
            cp_s[3].wait()
            _sum_store(items[3], selrows[3])
            return carry

        lax.fori_loop(0, npi // 4, quad_body, 0)

    return sc_k(stage, emb)


def _ln(x, g, b):
    n = x.shape[-1]
    mu = jnp.mean(x, axis=-1, keepdims=True)
    var = jnp.sum((x - mu) ** 2, axis=-1, keepdims=True) / (n - 1)
    sd = jnp.sqrt(var)
    return g * (x - mu) / (sd + 1e-3) + b


def _senc(x, w1, b1, w2, b2, g, b):
    h = jax.nn.relu(jnp.dot(x, w1, preferred_element_type=jnp.float32) + b1)
    h = jnp.dot(h, w2, preferred_element_type=jnp.float32) + b2
    return _ln(h + x, g, b)


def _dense_body(mcq_ref, mcs_ref, gw_ref, gb_ref, w1_ref, b1_ref, w2_ref,
                b2_ref, lng_ref, lnb_ref, wih_ref, whh_ref, brow_ref, out_ref):
    f32 = jnp.float32
    gw = gw_ref[...]
    gb = gb_ref[...]
    w1 = w1_ref[...]
    b1 = b1_ref[...]
    w2 = w2_ref[...]
    b2 = b2_ref[...]
    lng = lng_ref[...]
    lnb = lnb_ref[...]

    mcs = mcs_ref[...]
    s_gcn = jnp.tanh(jnp.dot(mcs, gw, preferred_element_type=f32) + gb)
    sn8 = jnp.concatenate([s_gcn[0:8], s_gcn[8:16]], axis=1)
    s_enc = _senc(sn8, w1, b1, w2, b2, lng, lnb)
    smask = (jax.lax.broadcasted_iota(jnp.int32, (8, 1), 0) < 5).astype(f32)
    sg = jnp.sum(s_enc * smask, axis=0, keepdims=True) * (1.0 / 5.0)

    mcq = mcq_ref[...]
    q_l = jnp.tanh(jnp.dot(mcq[:, :256], gw, preferred_element_type=f32) + gb)
    q_r = jnp.tanh(jnp.dot(mcq[:, 256:], gw, preferred_element_type=f32) + gb)
    qn = jnp.concatenate([q_l, q_r], axis=1)
    qx = _senc(qn, w1, b1, w2, b2, lng, lnb)

    wih = wih_ref[...]
    whh = whh_ref[...]
    brow = brow_ref[...]
    whh_a = whh[:, :256]
    whh_b = whh[:, 256:]

    qwih = jax.lax.dot_general(qx, wih, (((1,), (1,)), ((), ())),
                               preferred_element_type=f32) + brow
    supw = jax.lax.dot_general(sg, whh_b, (((1,), (1,)), ((), ())),
                               preferred_element_type=f32)

    bb = qx.shape[0]
    c = jnp.zeros((bb, 512), f32)
    h = qx
    for t in range(4):
        if t == 0:
            gates = qwih
        else:
            gates = qwih + supw + jax.lax.dot_general(
                h, whh_a, (((1,), (1,)), ((), ())), preferred_element_type=f32)
        gi = gates[:, 0:512]
        gf = gates[:, 512:1024]
        gg = gates[:, 1024:1536]
        go = gates[:, 1536:2048]
        c = jax.nn.sigmoid(gf) * c + jax.nn.sigmoid(gi) * jnp.tanh(gg)
        hc = jax.nn.sigmoid(go) * jnp.tanh(c)
        h = qx + hc[:, :256]

    qf = h / jnp.maximum(
        jnp.sqrt(jnp.sum(h * h, axis=1, keepdims=True)), 1e-12)
    sgn = sg / jnp.maximum(jnp.sqrt(jnp.sum(sg * sg)), 1e-12)
    out_ref[...] = jnp.sum(qf * sgn, axis=1)


def _dense_call(mcq, mcs16, gw, gb, w1, b1, w2, b2, lng, lnb, wih, whh, brow):
    B = mcq.shape[0]
    grid = (B // _BBLK,)
    full = lambda shape: pl.BlockSpec(shape, lambda i: (0,) * len(shape))
    return pl.pallas_call(
        _dense_body,
        grid=grid,
        in_specs=[
            pl.BlockSpec((_BBLK, 512), lambda i: (i, 0)),
            full((16, 256)),
            full((256, 128)), full((128,)),
            full((256, 512)), full((512,)),
            full((512, 256)), full((256,)),
            full((256,)), full((256,)),
            full((2048, 256)), full((2048, 512)), full((2048,)),
        ],
        out_specs=pl.BlockSpec((_BBLK,), lambda i: (i,)),
        out_shape=jax.ShapeDtypeStruct((B,), jnp.float32),
    )(mcq, mcs16, gw, gb, w1, b1, w2, b2, lng, lnb, wih, whh, brow)


def kernel(query, support, q_l_conn, q_l_deg, q_r_conn, q_r_deg, s_l_conn,
           s_l_deg, s_r_conn, s_r_deg, emb, gcn_w_W, gcn_w_b, gcn_b, se_w1,
           se_b1, se_w2, se_b2, se_ln_g, se_ln_b, lstm_wih, lstm_whh,
           lstm_bih, lstm_bhh):
    B = query.shape[0]
    FEW = support.shape[0]
    n = 2 * B + 2 * FEW
    npad = ((n + 4 * _NW - 1) // (4 * _NW)) * (4 * _NW)

    conn = jnp.concatenate(
        [q_l_conn, q_r_conn, s_l_conn, s_r_conn], axis=0)
    cids = jnp.concatenate(
        [query[:, 0], query[:, 1], support[:, 0], support[:, 1]], axis=0)
    eids = conn[:, :, 1].astype(jnp.int32)
    rids = conn[:, :, 0].astype(jnp.int32)
    cen8 = jnp.broadcast_to(cids.astype(jnp.int32)[:, None], (n, 8))
    pad8 = jnp.zeros((n, 8), jnp.int32)
    stage = jnp.concatenate(
        [eids, cen8, rids, pad8], axis=1)
    stage = jnp.pad(stage, ((0, npad - n), (0, 0)))

    mc = _sc_neighbor_call(stage, emb, npad)

    mcq = jnp.concatenate([mc[0:B], mc[B:2 * B]], axis=1)
    mcs16 = jnp.zeros((16, 256), jnp.float32)
    mcs16 = mcs16.at[0:5].set(mc[2 * B:2 * B + FEW])
    mcs16 = mcs16.at[8:13].set(mc[2 * B + FEW:2 * B + 2 * FEW])

    gb = gcn_w_b + gcn_b
    brow = lstm_bih + lstm_bhh
    return _dense_call(mcq, mcs16, gcn_w_W, gb, se_w1, se_b1, se_w2, se_b2,
                       se_ln_g, se_ln_b, lstm_wih, lstm_whh, brow)


# speedup vs baseline: 1.3028x; 1.3028x over previous
import functools

import jax
import jax.numpy as jnp
from jax import lax
from jax.experimental import pallas as pl
from jax.experimental.pallas import tpu as pltpu
from jax.experimental.pallas import tpu_sc as plsc

_D = 128
_NB = 64
_K = 32
_BBLK = 512
_NW = 32


def _shuf(x, perm):
    return lax.gather(
        x, perm[:, None],
        dimension_numbers=lax.GatherDimensionNumbers(
            offset_dims=(), collapsed_slice_dims=(0,), start_index_map=(0,)),
        slice_sizes=(1,),
        mode=lax.GatherScatterMode.PROMISE_IN_BOUNDS)


def _clean16(iota, k, v):
    for dist in (8, 4, 2, 1):
        perm = iota ^ dist
        pk, pv = _shuf(k, perm), _shuf(v, perm)
        up = (iota & dist) == 0
        c = k > pk
        bk = jnp.where(c, k, pk)
        bv = jnp.where(c, v, pv)
        sk = jnp.where(c, pk, k)
        sv = jnp.where(c, pv, v)
        k = jnp.where(up, bk, sk)
        v = jnp.where(up, bv, sv)
    return k, v


def _merge16(iota, ak, av, bk, bv):
    rbk, rbv = lax.rev(bk, (0,)), lax.rev(bv, (0,))
    c = ak > rbk
    hk = jnp.where(c, ak, rbk)
    hv = jnp.where(c, av, rbv)
    lk = jnp.where(c, rbk, ak)
    lv = jnp.where(c, rbv, av)
    hk, hv = _clean16(iota, hk, hv)
    lk, lv = _clean16(iota, lk, lv)
    return hk, hv, lk, lv


def _rsqrt_nr(x):
    xi = lax.bitcast_convert_type(x, jnp.int32)
    yi = jnp.int32(0x5F3759DF) - (xi >> 1)
    y = lax.bitcast_convert_type(yi, jnp.float32)
    for _ in range(3):
        y = y * (1.5 - 0.5 * x * y * y)
    return y


def _float_key(sim):
    b = lax.bitcast_convert_type(sim, jnp.int32)
    mask = b >> 31
    return b ^ (mask & jnp.int32(0x7FFFFFFF))


def _sc_neighbor_call(stage, emb, npad):
    npi = npad // _NW
    mesh = plsc.VectorSubcoreMesh(core_axis_name="c", subcore_axis_name="s")

    @functools.partial(
        pl.kernel, mesh=mesh,
        out_type=jax.ShapeDtypeStruct((npad, 2 * _D), jnp.float32),
        compiler_params=pltpu.CompilerParams(needs_layout_passes=False),
        scratch_types=(
            [pltpu.VMEM((144,), jnp.int32)] * 2 +
            [pltpu.VMEM((72, _D), jnp.float32)] * 2 +
            [pltpu.VMEM((_NB,), jnp.int32)] * 2 +
            [pltpu.VMEM((_NB, _D), jnp.float32)] * 2 +
            [
                pltpu.VMEM((_NB, 16), jnp.float32),
                pltpu.VMEM((_NB, 16), jnp.float32),
                pltpu.VMEM((2 * _D,), jnp.float32),
            ] +
            [pltpu.SemaphoreType.DMA] * 4
        ),
    )
    def sc_k(stage_hbm, emb_hbm, out_hbm,
             stage_0, stage_1, rows_0, rows_1,
             selidx_0, selidx_1, selrows_0, selrows_1,
             p_buf, q_buf, out_v,
             sem_r0, sem_r1, sem_s0, sem_s1):
        wid = lax.axis_index("s") * 2 + lax.axis_index("c")
        base = wid * npi
        iota = lax.iota(jnp.int32, 16)
        jvs = [iota + 16 * g for g in range(4)]
        zero = jnp.zeros((16,), jnp.float32)
        inv = jnp.float32(1.0 / _K)

        def _tree16(vecs):
            dist = 1
            while len(vecs) > 1:
                nxt = []
                for t in range(len(vecs) // 2):
                    a, b = vecs[2 * t], vecs[2 * t + 1]
                    sa = a + _shuf(a, iota ^ dist)
                    sb = b + _shuf(b, iota ^ dist)
                    nxt.append(jnp.where((iota & dist) == 0, sa, sb))
                vecs = nxt
                dist *= 2
            return vecs[0]

        def _sims_sort_sel(rows_v, stage_v, selidx_v):
            cv = jnp.full((16,), 64, jnp.int32)
            cen = [plsc.load_gather(rows_v, [cv, iota + cc * 16])
                   for cc in range(8)]

            def jbody(jo, carry):
                for ju in range(2):
                    j = jo * 2 + ju
                    jv = jnp.full((16,), j, jnp.int32)
                    e = [plsc.load_gather(rows_v, [jv, iota + cc * 16])
                         for cc in range(8)]
                    p = cen[0] * e[0]
                    q = e[0] * e[0]
                    for cc in range(1, 8):
                        p = p + cen[cc] * e[cc]
                        q = q + e[cc] * e[cc]
                    plsc.store_scatter(p_buf, [jv, iota], p)
                    plsc.store_scatter(q_buf, [jv, iota], q)
                return carry

            lax.fori_loop(0, _NB // 2, jbody, 0)

            keys, vals = [], []
            for g in range(4):
                dots = _tree16([p_buf[g * 16 + jj] for jj in range(16)])
                nrms = _tree16([q_buf[g * 16 + jj] for jj in range(16)])
                sim = dots * _rsqrt_nr(nrms)
                sk = _float_key(sim) - jvs[g]
                k_, v_ = plsc.sort_key_val(sk, jvs[g], descending=True)
                keys.append(k_)
                vals.append(v_)

            a0k, a0v, a1k, a1v = _merge16(iota, keys[0], vals[0],
                                          keys[1], vals[1])
            b0k, b0v, b1k, b1v = _merge16(iota, keys[2], vals[2],
                                          keys[3], vals[3])
            r0v = lax.rev(b1v, (0,))
            r1v = lax.rev(b0v, (0,))
            h0v = jnp.where(a0k > lax.rev(b1k, (0,)), a0v, r0v)
            h1v = jnp.where(a1k > lax.rev(b0k, (0,)), a1v, r1v)

            esel0 = plsc.load_gather(stage_v, [h0v])
            esel1 = plsc.load_gather(stage_v, [h1v])
            rsel0 = plsc.load_gather(stage_v, [h0v + 72])
            rsel1 = plsc.load_gather(stage_v, [h1v + 72])
            selidx_v[pl.ds(0, 16)] = esel0
            selidx_v[pl.ds(16, 16)] = esel1
            selidx_v[pl.ds(32, 16)] = rsel0
            selidx_v[pl.ds(48, 16)] = rsel1

        def _sum_store(item, selrows_v):
            def sbody2(jo, accs):
                new = list(accs)
                for ju in range(4):
                    jv = jnp.full((16,), jo * 4 + ju, jnp.int32)
                    for cc in range(8):
                        colidx = iota + cc * 16
                        ecol = plsc.load_gather(selrows_v, [jv, colidx])
                        rcol = plsc.load_gather(selrows_v, [jv + 32, colidx])
                        new[cc] = new[cc] + ecol
                        new[8 + cc] = new[8 + cc] + rcol
                return tuple(new)

            sums = lax.fori_loop(0, 8, sbody2, (zero,) * 16)
            for cc in range(8):
                out_v[pl.ds(cc * 16, 16)] = sums[8 + cc] * inv
                out_v[pl.ds(_D + cc * 16, 16)] = sums[cc] * inv
            pltpu.sync_copy(out_v, out_hbm.at[item])

        def pair_body(t, carry):
            i = base + 2 * t
            j = i + 1
            pltpu.sync_copy(stage_hbm.at[i], stage_0)
            cp_r0 = pltpu.async_copy(
                emb_hbm.at[stage_0.at[pl.ds(0, 72)]], rows_0, sem_r0)
            pltpu.sync_copy(stage_hbm.at[j], stage_1)
            cp_r1 = pltpu.async_copy(
                emb_hbm.at[stage_1.at[pl.ds(0, 72)]], rows_1, sem_r1)
            cp_r0.wait()
            _sims_sort_sel(rows_0, stage_0, selidx_0)
            cp_s0 = pltpu.async_copy(emb_hbm.at[selidx_0], selrows_0, sem_s0)
            cp_r1.wait()
            _sims_sort_sel(rows_1, stage_1, selidx_1)
            cp_s1 = pltpu.async_copy(emb_hbm.at[selidx_1], selrows_1, sem_s1)
            cp_s0.wait()
            _sum_store(i, selrows_0)
            cp_s1.wait()
            _sum_store(j, selrows_1)
            return carry

        lax.fori_loop(0, npi // 2, pair_body, 0)

    return sc_k(stage, emb)


def _ln(x, g, b):
    n = x.shape[-1]
    mu = jnp.mean(x, axis=-1, keepdims=True)
    var = jnp.sum((x - mu) ** 2, axis=-1, keepdims=True) / (n - 1)
    sd = jnp.sqrt(var)
    return g * (x - mu) / (sd + 1e-3) + b


def _senc(x, w1, b1, w2, b2, g, b):
    h = jax.nn.relu(jnp.dot(x, w1, preferred_element_type=jnp.float32) + b1)
    h = jnp.dot(h, w2, preferred_element_type=jnp.float32) + b2
    return _ln(h + x, g, b)


def _dense_body(mcq_ref, mcs_ref, gw_ref, gb_ref, w1_ref, b1_ref, w2_ref,
                b2_ref, lng_ref, lnb_ref, wih_ref, whh_ref, brow_ref, out_ref):
    f32 = jnp.float32
    gw = gw_ref[...]
    gb = gb_ref[...]
    w1 = w1_ref[...]
    b1 = b1_ref[...]
    w2 = w2_ref[...]
    b2 = b2_ref[...]
    lng = lng_ref[...]
    lnb = lnb_ref[...]

    mcs = mcs_ref[...]
    s_gcn = jnp.tanh(jnp.dot(mcs, gw, preferred_element_type=f32) + gb)
    sn8 = jnp.concatenate([s_gcn[0:8], s_gcn[8:16]], axis=1)
    s_enc = _senc(sn8, w1, b1, w2, b2, lng, lnb)
    smask = (jax.lax.broadcasted_iota(jnp.int32, (8, 1), 0) < 5).astype(f32)
    sg = jnp.sum(s_enc * smask, axis=0, keepdims=True) * (1.0 / 5.0)

    mcq = mcq_ref[...]
    q_l = jnp.tanh(jnp.dot(mcq[:, :256], gw, preferred_element_type=f32) + gb)
    q_r = jnp.tanh(jnp.dot(mcq[:, 256:], gw, preferred_element_type=f32) + gb)
    qn = jnp.concatenate([q_l, q_r], axis=1)
    qx = _senc(qn, w1, b1, w2, b2, lng, lnb)

    wih = wih_ref[...]
    whh = whh_ref[...]
    brow = brow_ref[...]
    whh_a = whh[:, :256]
    whh_b = whh[:, 256:]

    qwih = jax.lax.dot_general(qx, wih, (((1,), (1,)), ((), ())),
                               preferred_element_type=f32) + brow
    supw = jax.lax.dot_general(sg, whh_b, (((1,), (1,)), ((), ())),
                               preferred_element_type=f32)

    bb = qx.shape[0]
    c = jnp.zeros((bb, 512), f32)
    h = qx
    for t in range(4):
        if t == 0:
            gates = qwih
        else:
            gates = qwih + supw + jax.lax.dot_general(
                h, whh_a, (((1,), (1,)), ((), ())), preferred_element_type=f32)
        gi = gates[:, 0:512]
        gf = gates[:, 512:1024]
        gg = gates[:, 1024:1536]
        go = gates[:, 1536:2048]
        c = jax.nn.sigmoid(gf) * c + jax.nn.sigmoid(gi) * jnp.tanh(gg)
        hc = jax.nn.sigmoid(go) * jnp.tanh(c)
        h = qx + hc[:, :256]

    qf = h / jnp.maximum(
        jnp.sqrt(jnp.sum(h * h, axis=1, keepdims=True)), 1e-12)
    sgn = sg / jnp.maximum(jnp.sqrt(jnp.sum(sg * sg)), 1e-12)
    out_ref[...] = jnp.sum(qf * sgn, axis=1)


def _dense_call(mcq, mcs16, gw, gb, w1, b1, w2, b2, lng, lnb, wih, whh, brow):
    B = mcq.shape[0]
    grid = (B // _BBLK,)
    full = lambda shape: pl.BlockSpec(shape, lambda i: (0,) * len(shape))
    return pl.pallas_call(
        _dense_body,
        grid=grid,
        in_specs=[
            pl.BlockSpec((_BBLK, 512), lambda i: (i, 0)),
            full((16, 256)),
            full((256, 128)), full((128,)),
            full((256, 512)), full((512,)),
            full((512, 256)), full((256,)),
            full((256,)), full((256,)),
            full((2048, 256)), full((2048, 512)), full((2048,)),
        ],
        out_specs=pl.BlockSpec((_BBLK,), lambda i: (i,)),
        out_shape=jax.ShapeDtypeStruct((B,), jnp.float32),
    )(mcq, mcs16, gw, gb, w1, b1, w2, b2, lng, lnb, wih, whh, brow)


def kernel(query, support, q_l_conn, q_l_deg, q_r_conn, q_r_deg, s_l_conn,
           s_l_deg, s_r_conn, s_r_deg, emb, gcn_w_W, gcn_w_b, gcn_b, se_w1,
           se_b1, se_w2, se_b2, se_ln_g, se_ln_b, lstm_wih, lstm_whh,
           lstm_bih, lstm_bhh):
    B = query.shape[0]
    FEW = support.shape[0]
    n = 2 * B + 2 * FEW
    npad = ((n + 2 * _NW - 1) // (2 * _NW)) * (2 * _NW)

    conn = jnp.concatenate(
        [q_l_conn, q_r_conn, s_l_conn, s_r_conn], axis=0)
    cids = jnp.concatenate(
        [query[:, 0], query[:, 1], support[:, 0], support[:, 1]], axis=0)
    eids = conn[:, :, 1].astype(jnp.int32)
    rids = conn[:, :, 0].astype(jnp.int32)
    cen8 = jnp.broadcast_to(cids.astype(jnp.int32)[:, None], (n, 8))
    pad8 = jnp.zeros((n, 8), jnp.int32)
    stage = jnp.concatenate(
        [eids, cen8, rids, pad8], axis=1)
    stage = jnp.pad(stage, ((0, npad - n), (0, 0)))

    mc = _sc_neighbor_call(stage, emb, npad)

    mcq = jnp.concatenate([mc[0:B], mc[B:2 * B]], axis=1)
    mcs16 = jnp.zeros((16, 256), jnp.float32)
    mcs16 = mcs16.at[0:5].set(mc[2 * B:2 * B + FEW])
    mcs16 = mcs16.at[8:13].set(mc[2 * B + FEW:2 * B + 2 * FEW])

    gb = gcn_w_b + gcn_b
    brow = lstm_bih + lstm_bhh
    return _dense_call(mcq, mcs16, gcn_w_W, gb, se_w1, se_b1, se_w2, se_b2,
                       se_ln_g, se_ln_b, lstm_wih, lstm_whh, brow)


# speedup vs baseline: 1.6339x; 1.2541x over previous
import functools

import jax
import jax.numpy as jnp
from jax import lax
from jax.experimental import pallas as pl
from jax.experimental.pallas import tpu as pltpu
from jax.experimental.pallas import tpu_sc as plsc

_D = 128
_NB = 64
_K = 32
_BBLK = 512
_NW = 32


def _shuf(x, perm):
    return lax.gather(
        x, perm[:, None],
        dimension_numbers=lax.GatherDimensionNumbers(
            offset_dims=(), collapsed_slice_dims=(0,), start_index_map=(0,)),
        slice_sizes=(1,),
        mode=lax.GatherScatterMode.PROMISE_IN_BOUNDS)


def _clean16(iota, k, v):
    for dist in (8, 4, 2, 1):
        perm = iota ^ dist
        pk, pv = _shuf(k, perm), _shuf(v, perm)
        up = (iota & dist) == 0
        c = k > pk
        bk = jnp.where(c, k, pk)
        bv = jnp.where(c, v, pv)
        sk = jnp.where(c, pk, k)
        sv = jnp.where(c, pv, v)
        k = jnp.where(up, bk, sk)
        v = jnp.where(up, bv, sv)
    return k, v


def _merge16(iota, ak, av, bk, bv):
    rbk, rbv = lax.rev(bk, (0,)), lax.rev(bv, (0,))
    c = ak > rbk
    hk = jnp.where(c, ak, rbk)
    hv = jnp.where(c, av, rbv)
    lk = jnp.where(c, rbk, ak)
    lv = jnp.where(c, rbv, av)
    hk, hv = _clean16(iota, hk, hv)
    lk, lv = _clean16(iota, lk, lv)
    return hk, hv, lk, lv


def _rsqrt_nr(x):
    xi = lax.bitcast_convert_type(x, jnp.int32)
    yi = jnp.int32(0x5F3759DF) - (xi >> 1)
    y = lax.bitcast_convert_type(yi, jnp.float32)
    for _ in range(3):
        y = y * (1.5 - 0.5 * x * y * y)
    return y


def _float_key(sim):
    b = lax.bitcast_convert_type(sim, jnp.int32)
    mask = b >> 31
    return b ^ (mask & jnp.int32(0x7FFFFFFF))


def _sc_neighbor_call(stage, emb, npad):
    npi = npad // _NW
    mesh = plsc.VectorSubcoreMesh(core_axis_name="c", subcore_axis_name="s")

    @functools.partial(
        pl.kernel, mesh=mesh,
        out_type=jax.ShapeDtypeStruct((npad, 2 * _D), jnp.float32),
        compiler_params=pltpu.CompilerParams(needs_layout_passes=False),
        scratch_types=(
            [pltpu.VMEM((144,), jnp.int32)] * 2 +
            [pltpu.VMEM((72, _D), jnp.float32)] * 2 +
            [pltpu.VMEM((_K,), jnp.int32)] * 2 +
            [pltpu.VMEM((_K, _D), jnp.float32)] * 2 +
            [pltpu.VMEM((_K,), jnp.int32)] * 2 +
            [
                pltpu.VMEM((_NB, 16), jnp.float32),
                pltpu.VMEM((_NB, 16), jnp.float32),
                pltpu.VMEM((2 * _D,), jnp.float32),
            ] +
            [pltpu.SemaphoreType.DMA] * 4
        ),
    )
    def sc_k(stage_hbm, emb_hbm, out_hbm,
             stage_0, stage_1, rows_0, rows_1,
             selidx_0, selidx_1, selrows_0, selrows_1, hsel_0, hsel_1,
             p_buf, q_buf, out_v,
             sem_r0, sem_r1, sem_s0, sem_s1):
        wid = lax.axis_index("s") * 2 + lax.axis_index("c")
        base = wid * npi
        iota = lax.iota(jnp.int32, 16)
        jvs = [iota + 16 * g for g in range(4)]
        zero = jnp.zeros((16,), jnp.float32)
        inv = jnp.float32(1.0 / _K)

        def _tree16(vecs):
            dist = 1
            while len(vecs) > 1:
                nxt = []
                for t in range(len(vecs) // 2):
                    a, b = vecs[2 * t], vecs[2 * t + 1]
                    sa = a + _shuf(a, iota ^ dist)
                    sb = b + _shuf(b, iota ^ dist)
                    nxt.append(jnp.where((iota & dist) == 0, sa, sb))
                vecs = nxt
                dist *= 2
            return vecs[0]

        def _sims_sort_sel(rows_v, stage_v, selidx_v, hsel_v):
            cv = jnp.full((16,), 64, jnp.int32)
            cen = [plsc.load_gather(rows_v, [cv, iota + cc * 16])
                   for cc in range(8)]

            def jbody(jo, carry):
                for ju in range(2):
                    j = jo * 2 + ju
                    jv = jnp.full((16,), j, jnp.int32)
                    e = [plsc.load_gather(rows_v, [jv, iota + cc * 16])
                         for cc in range(8)]
                    p = cen[0] * e[0]
                    q = e[0] * e[0]
                    for cc in range(1, 8):
                        p = p + cen[cc] * e[cc]
                        q = q + e[cc] * e[cc]
                    plsc.store_scatter(p_buf, [jv, iota], p)
                    plsc.store_scatter(q_buf, [jv, iota], q)
                return carry

            lax.fori_loop(0, _NB // 2, jbody, 0)

            keys, vals = [], []
            for g in range(4):
                dots = _tree16([p_buf[g * 16 + jj] for jj in range(16)])
                nrms = _tree16([q_buf[g * 16 + jj] for jj in range(16)])
                sim = dots * _rsqrt_nr(nrms)
                sk = _float_key(sim) - jvs[g]
                k_, v_ = plsc.sort_key_val(sk, jvs[g], descending=True)
                keys.append(k_)
                vals.append(v_)

            a0k, a0v, a1k, a1v = _merge16(iota, keys[0], vals[0],
                                          keys[1], vals[1])
            b0k, b0v, b1k, b1v = _merge16(iota, keys[2], vals[2],
                                          keys[3], vals[3])
            r0v = lax.rev(b1v, (0,))
            r1v = lax.rev(b0v, (0,))
            h0v = jnp.where(a0k > lax.rev(b1k, (0,)), a0v, r0v)
            h1v = jnp.where(a1k > lax.rev(b0k, (0,)), a1v, r1v)

            rsel0 = plsc.load_gather(stage_v, [h0v + 72])
            rsel1 = plsc.load_gather(stage_v, [h1v + 72])
            selidx_v[pl.ds(0, 16)] = rsel0
            selidx_v[pl.ds(16, 16)] = rsel1
            hsel_v[pl.ds(0, 16)] = h0v
            hsel_v[pl.ds(16, 16)] = h1v

        def _sum_store(item, rows_v, selrows_v, hsel_v):
            hv0 = hsel_v[pl.ds(0, 16)]
            hv1 = hsel_v[pl.ds(16, 16)]

            def sbody2(jo, accs):
                new = list(accs)
                jj = jnp.full((16,), jo, jnp.int32)
                rv0 = _shuf(hv0, jj)
                rv1 = _shuf(hv1, jj)
                for ju in range(2):
                    jv = jnp.full((16,), jo * 2 + ju, jnp.int32)
                    for cc in range(8):
                        colidx = iota + cc * 16
                        rcol = plsc.load_gather(selrows_v, [jv, colidx])
                        new[8 + cc] = new[8 + cc] + rcol
                for cc in range(8):
                    colidx = iota + cc * 16
                    e0 = plsc.load_gather(rows_v, [rv0, colidx])
                    e1 = plsc.load_gather(rows_v, [rv1, colidx])
                    new[cc] = new[cc] + e0 + e1
                return tuple(new)

            sums = lax.fori_loop(0, 16, sbody2, (zero,) * 16)
            for cc in range(8):
                out_v[pl.ds(cc * 16, 16)] = sums[8 + cc] * inv
                out_v[pl.ds(_D + cc * 16, 16)] = sums[cc] * inv
            pltpu.sync_copy(out_v, out_hbm.at[item])

        def pair_body(t, carry):
            i = base + 2 * t
            j = i + 1
            pltpu.sync_copy(stage_hbm.at[i], stage_0)
            cp_r0 = pltpu.async_copy(
                emb_hbm.at[stage_0.at[pl.ds(0, 72)]], rows_0, sem_r0)
            pltpu.sync_copy(stage_hbm.at[j], stage_1)
            cp_r1 = pltpu.async_copy(
                emb_hbm.at[stage_1.at[pl.ds(0, 72)]], rows_1, sem_r1)
            cp_r0.wait()
            _sims_sort_sel(rows_0, stage_0, selidx_0, hsel_0)
            cp_s0 = pltpu.async_copy(emb_hbm.at[selidx_0], selrows_0, sem_s0)
            cp_r1.wait()
            _sims_sort_sel(rows_1, stage_1, selidx_1, hsel_1)
            cp_s1 = pltpu.async_copy(emb_hbm.at[selidx_1], selrows_1, sem_s1)
            cp_s0.wait()
            _sum_store(i, rows_0, selrows_0, hsel_0)
            cp_s1.wait()
            _sum_store(j, rows_1, selrows_1, hsel_1)
            return carry

        lax.fori_loop(0, npi // 2, pair_body, 0)

    return sc_k(stage, emb)


def _ln(x, g, b):
    n = x.shape[-1]
    mu = jnp.mean(x, axis=-1, keepdims=True)
    var = jnp.sum((x - mu) ** 2, axis=-1, keepdims=True) / (n - 1)
    sd = jnp.sqrt(var)
    return g * (x - mu) / (sd + 1e-3) + b


def _senc(x, w1, b1, w2, b2, g, b):
    h = jax.nn.relu(jnp.dot(x, w1, preferred_element_type=jnp.float32) + b1)
    h = jnp.dot(h, w2, preferred_element_type=jnp.float32) + b2
    return _ln(h + x, g, b)


def _dense_body(mcq_ref, mcs_ref, gw_ref, gb_ref, w1_ref, b1_ref, w2_ref,
                b2_ref, lng_ref, lnb_ref, wih_ref, whh_ref, brow_ref, out_ref):
    f32 = jnp.float32
    gw = gw_ref[...]
    gb = gb_ref[...]
    w1 = w1_ref[...]
    b1 = b1_ref[...]
    w2 = w2_ref[...]
    b2 = b2_ref[...]
    lng = lng_ref[...]
    lnb = lnb_ref[...]

    mcs = mcs_ref[...]
    s_gcn = jnp.tanh(jnp.dot(mcs, gw, preferred_element_type=f32) + gb)
    sn8 = jnp.concatenate([s_gcn[0:8], s_gcn[8:16]], axis=1)
    s_enc = _senc(sn8, w1, b1, w2, b2, lng, lnb)
    smask = (jax.lax.broadcasted_iota(jnp.int32, (8, 1), 0) < 5).astype(f32)
    sg = jnp.sum(s_enc * smask, axis=0, keepdims=True) * (1.0 / 5.0)

    mcq = mcq_ref[...]
    q_l = jnp.tanh(jnp.dot(mcq[:, :256], gw, preferred_element_type=f32) + gb)
    q_r = jnp.tanh(jnp.dot(mcq[:, 256:], gw, preferred_element_type=f32) + gb)
    qn = jnp.concatenate([q_l, q_r], axis=1)
    qx = _senc(qn, w1, b1, w2, b2, lng, lnb)

    wih = wih_ref[...]
    whh = whh_ref[...]
    brow = brow_ref[...]
    whh_a = whh[:, :256]
    whh_b = whh[:, 256:]

    qwih = jax.lax.dot_general(qx, wih, (((1,), (1,)), ((), ())),
                               preferred_element_type=f32) + brow
    supw = jax.lax.dot_general(sg, whh_b, (((1,), (1,)), ((), ())),
                               preferred_element_type=f32)

    bb = qx.shape[0]
    c = jnp.zeros((bb, 512), f32)
    h = qx
    for t in range(4):
        if t == 0:
            gates = qwih
        else:
            gates = qwih + supw + jax.lax.dot_general(
                h, whh_a, (((1,), (1,)), ((), ())), preferred_element_type=f32)
        gi = gates[:, 0:512]
        gf = gates[:, 512:1024]
        gg = gates[:, 1024:1536]
        go = gates[:, 1536:2048]
        c = jax.nn.sigmoid(gf) * c + jax.nn.sigmoid(gi) * jnp.tanh(gg)
        hc = jax.nn.sigmoid(go) * jnp.tanh(c)
        h = qx + hc[:, :256]

    qf = h / jnp.maximum(
        jnp.sqrt(jnp.sum(h * h, axis=1, keepdims=True)), 1e-12)
    sgn = sg / jnp.maximum(jnp.sqrt(jnp.sum(sg * sg)), 1e-12)
    out_ref[...] = jnp.sum(qf * sgn, axis=1)


def _dense_call(mcq, mcs16, gw, gb, w1, b1, w2, b2, lng, lnb, wih, whh, brow):
    B = mcq.shape[0]
    grid = (B // _BBLK,)
    full = lambda shape: pl.BlockSpec(shape, lambda i: (0,) * len(shape))
    return pl.pallas_call(
        _dense_body,
        grid=grid,
        in_specs=[
            pl.BlockSpec((_BBLK, 512), lambda i: (i, 0)),
            full((16, 256)),
            full((256, 128)), full((128,)),
            full((256, 512)), full((512,)),
            full((512, 256)), full((256,)),
            full((256,)), full((256,)),
            full((2048, 256)), full((2048, 512)), full((2048,)),
        ],
        out_specs=pl.BlockSpec((_BBLK,), lambda i: (i,)),
        out_shape=jax.ShapeDtypeStruct((B,), jnp.float32),
    )(mcq, mcs16, gw, gb, w1, b1, w2, b2, lng, lnb, wih, whh, brow)


def kernel(query, support, q_l_conn, q_l_deg, q_r_conn, q_r_deg, s_l_conn,
           s_l_deg, s_r_conn, s_r_deg, emb, gcn_w_W, gcn_w_b, gcn_b, se_w1,
           se_b1, se_w2, se_b2, se_ln_g, se_ln_b, lstm_wih, lstm_whh,
           lstm_bih, lstm_bhh):
    B = query.shape[0]
    FEW = support.shape[0]
    n = 2 * B + 2 * FEW
    npad = ((n + 2 * _NW - 1) // (2 * _NW)) * (2 * _NW)

    conn = jnp.concatenate(
        [q_l_conn, q_r_conn, s_l_conn, s_r_conn], axis=0)
    cids = jnp.concatenate(
        [query[:, 0], query[:, 1], support[:, 0], support[:, 1]], axis=0)
    eids = conn[:, :, 1].astype(jnp.int32)
    rids = conn[:, :, 0].astype(jnp.int32)
    cen8 = jnp.broadcast_to(cids.astype(jnp.int32)[:, None], (n, 8))
    pad8 = jnp.zeros((n, 8), jnp.int32)
    stage = jnp.concatenate(
        [eids, cen8, rids, pad8], axis=1)
    stage = jnp.pad(stage, ((0, npad - n), (0, 0)))

    mc = _sc_neighbor_call(stage, emb, npad)

    mcq = jnp.concatenate([mc[0:B], mc[B:2 * B]], axis=1)
    mcs16 = jnp.zeros((16, 256), jnp.float32)
    mcs16 = mcs16.at[0:5].set(mc[2 * B:2 * B + FEW])
    mcs16 = mcs16.at[8:13].set(mc[2 * B + FEW:2 * B + 2 * FEW])

    gb = gcn_w_b + gcn_b
    brow = lstm_bih + lstm_bhh
    return _dense_call(mcq, mcs16, gcn_w_W, gb, se_w1, se_b1, se_w2, se_b2,
                       se_ln_g, se_ln_b, lstm_wih, lstm_whh, brow)
